# baseline (speedup 1.0000x reference)
import functools

import jax
import jax.numpy as jnp
from jax import lax
from jax.experimental import pallas as pl
from jax.experimental.pallas import tpu as pltpu
from jax.experimental.pallas import tpu_sc as plsc

SOC_G = 17
SOC_L = 23

_B = 4
_S = 2048
_N = _B * _S
_H = 1024
_G = 1024
_LMAX = 16

_NC = 2
_NS = 16
_NW = _NC * _NS
_TPW = _N // _NW
_T = 32
_NCH = _TPW // _T
_K = 8
_R = 72
_NR = _N // 128
_RPB = _S // 128


def _f32(x):
    return x.astype(jnp.float32)


def _tc_body(ids_ref, lmf_ref, g_ref, lv_ref, w_ref, b_ref,
             extra_ref, srow_ref, dtok_ref):
    f32 = jnp.float32
    ids = ids_ref[...]
    lmf = lmf_ref[...]

    w = w_ref[...]
    bias = b_ref[...]
    pg = jnp.dot(g_ref[...], w, preferred_element_type=f32) + bias
    plv = jnp.dot(lv_ref[...], w, preferred_element_type=f32) + bias
    extra_ref[...] = jnp.concatenate([pg, plv, pg], axis=0)

    il = _f32(ids == SOC_L)
    ig = _f32(ids == SOC_G)

    m_i = lax.broadcasted_iota(jnp.int32, (128, 128), 0)
    m_j = lax.broadcasted_iota(jnp.int32, (128, 128), 1)
    m_incl = _f32(m_i <= m_j)
    m_last = _f32(m_i == 127)

    r_i = lax.broadcasted_iota(jnp.int32, (_NR, _NR), 0)
    c_i = lax.broadcasted_iota(jnp.int32, (_NR, _NR), 1)

    incl = jnp.dot(il, m_incl, preferred_element_type=f32)
    bsf = jnp.dot(incl, m_last, preferred_element_type=f32)
    m_carry = _f32((r_i // _RPB == c_i // _RPB) & (c_i < r_i))
    carry = jnp.dot(m_carry, bsf, preferred_element_type=f32)
    rank = incl + carry - 1.0

    sel = _f32(lax.broadcasted_iota(jnp.int32, (_NR, _B), 0) // _RPB
               == lax.broadcasted_iota(jnp.int32, (_NR, _B), 1))
    ones16_128 = jnp.ones((_LMAX, 128), f32)
    nv = jnp.dot(sel, jnp.dot(lmf, ones16_128, preferred_element_type=f32),
                 preferred_element_type=f32)
    inject = (il > 0.0) & (rank < nv)

    l_i = lax.broadcasted_iota(jnp.int32, (_LMAX, _LMAX), 0)
    l_j = lax.broadcasted_iota(jnp.int32, (_LMAX, _LMAX), 1)
    cm = jnp.dot(lmf, _f32(l_i <= l_j), preferred_element_type=f32)
    jidx = _f32(lax.broadcasted_iota(jnp.int32, (_B, _LMAX), 1))
    st_cols = [
        jnp.sum(lmf * _f32(cm == (r + 1)) * jidx, axis=1, keepdims=True)
        for r in range(_LMAX)
    ]
    st = jnp.concatenate(st_cols, axis=1)
    st64 = jnp.dot(sel, st, preferred_element_type=f32)

    rankc = jnp.clip(rank, 0.0, float(_LMAX - 1))
    slot = jnp.zeros_like(rank)
    for r in range(_LMAX):
        slot = slot + jnp.where(rankc == float(r), st64[:, r:r + 1], 0.0)

    bidx = _f32(lax.broadcasted_iota(jnp.int32, (_NR, 1), 0) // _RPB)
    extrarow = jnp.where(ig > 0.0, bidx,
                         float(_B) + bidx * float(_LMAX) + slot)
    ovr = ig + _f32(inject)

    incl2 = jnp.dot(ovr, m_incl, preferred_element_type=f32)
    bsf2 = jnp.dot(incl2, m_last, preferred_element_type=f32)
    m_pair = _f32((c_i == r_i - 1) & (r_i % 2 == 1))
    carry2 = jnp.dot(m_pair, bsf2, preferred_element_type=f32)
    oc = incl2 + carry2

    pos = _f32((lax.broadcasted_iota(jnp.int32, (_NR, 128), 0) % 2) * 128
               + lax.broadcasted_iota(jnp.int32, (_NR, 128), 1))
    ods, vls = [], []
    for k in range(_K):
        hit = ovr * _f32(oc == float(k + 1))
        ods.append(jnp.sum(hit * pos, axis=1, keepdims=True))
        vls.append(jnp.sum(hit * extrarow, axis=1, keepdims=True))
    ords = jnp.concatenate(ods, axis=1)
    vals = jnp.concatenate(vls, axis=1)

    pair = _f32(lax.broadcasted_iota(jnp.int32, (_NW, _NR), 1) // 2
                == lax.broadcasted_iota(jnp.int32, (_NW, _NR), 0))
    ordc = jnp.dot(pair, ords, preferred_element_type=f32)
    valc = jnp.dot(pair, vals, preferred_element_type=f32)
    cntc = jnp.dot(pair, bsf2[:, 0:_K], preferred_element_type=f32)

    kio = _f32(lax.broadcasted_iota(jnp.int32, (_NW, _K), 1))
    wio = _f32(lax.broadcasted_iota(jnp.int32, (_NW, _K), 0))
    valid = kio < cntc
    srow_ref[...] = jnp.where(valid, valc, 0.0).astype(jnp.int32)
    dtok_ref[...] = jnp.where(
        valid, wio * float(_TPW) + ordc, 0.0).astype(jnp.int32)


def _tc_bookkeeping(ids2d, lmf, gvec, lvec, proj_w, proj_b2d):
    return pl.pallas_call(
        _tc_body,
        out_shape=(
            jax.ShapeDtypeStruct((_R, _H), jnp.float32),
            jax.ShapeDtypeStruct((_NW, _K), jnp.int32),
            jax.ShapeDtypeStruct((_NW, _K), jnp.int32),
        ),
    )(ids2d, lmf, gvec, lvec, proj_w, proj_b2d)


@functools.cache
def _make_sc_gather():
    mesh = plsc.VectorSubcoreMesh(core_axis_name="c", subcore_axis_name="s",
                                  num_cores=_NC, num_subcores=_NS)

    @functools.partial(
        pl.kernel,
        out_type=jax.ShapeDtypeStruct((_N, _H), jnp.float32),
        mesh=mesh,
        scratch_types=[
            pltpu.VMEM((_NCH, _T), jnp.int32),
            pltpu.VMEM((_T, _H), jnp.float32),
            pltpu.VMEM((_T, _H), jnp.float32),
            pltpu.VMEM((_T, _H), jnp.float32),
            pltpu.VMEM((_K,), jnp.int32),
            pltpu.VMEM((_K,), jnp.int32),
            pltpu.VMEM((_K, _H), jnp.float32),
            pltpu.SemaphoreType.DMA,
            pltpu.SemaphoreType.DMA,
            pltpu.SemaphoreType.DMA,
            pltpu.SemaphoreType.DMA,
            pltpu.SemaphoreType.DMA,
            pltpu.SemaphoreType.DMA,
            pltpu.SemaphoreType.DMA,
        ],
    )
    def sc_gather(ids_hbm, emb_hbm, extra_hbm, srow_hbm, dtok_hbm, out_hbm,
                  idx_v, buf0, buf1, buf2, srow_v, dtok_v, stag_v,
                  gs0, gs1, gs2, ss0, ss1, ss2, osem):
        wid = lax.axis_index("s") * _NC + lax.axis_index("c")
        base = wid * _TPW
        bufs = (buf0, buf1, buf2)
        gsems = (gs0, gs1, gs2)
        ssems = (ss0, ss1, ss2)
        nbuf = 3
        lag = 2
        pltpu.sync_copy(ids_hbm.at[wid], idx_v)
        hrow = hdtk = None
        hg = [None] * _NCH
        hs = [None] * _NCH
        scatter_waited = [False] * _NCH
        for c in range(_NCH + lag):
            if c < _NCH:
                b = c % nbuf
                if c >= nbuf:
                    hs[c - nbuf].wait()
                    scatter_waited[c - nbuf] = True
                hg[c] = pltpu.async_copy(
                    emb_hbm.at[idx_v.at[c]], bufs[b], gsems[b])
            if c == 0:
                hrow = pltpu.async_copy(srow_hbm.at[wid], srow_v, osem)
                hdtk = pltpu.async_copy(dtok_hbm.at[wid], dtok_v, osem)
            d = c - lag
            if d >= 0:
                hg[d].wait()
                hs[d] = pltpu.async_copy(
                    bufs[d % nbuf], out_hbm.at[pl.ds(base + d * _T, _T)],
                    ssems[d % nbuf])
        hrow.wait()
        hdtk.wait()
        hstag = pltpu.async_copy(extra_hbm.at[srow_v], stag_v, osem)
        for d in range(_NCH):
            if not scatter_waited[d]:
                hs[d].wait()
        hstag.wait()
        pltpu.async_copy(stag_v, out_hbm.at[dtok_v], osem).wait()

    return sc_gather


def kernel(input_ids, attention_mask, global_vec, local_vecs_padded,
           local_mask, emb_weight, proj_w, proj_b):
    del attention_mask
    ids = input_ids.astype(jnp.int32)

    extra_tab, src_rows, dst_tok = _tc_bookkeeping(
        ids.reshape(_NR, 128), local_mask.astype(jnp.float32), global_vec,
        local_vecs_padded.reshape(_B * _LMAX, _G), proj_w,
        proj_b.reshape(1, _H))

    out = _make_sc_gather()(ids.reshape(_NW, _NCH, _T), emb_weight, extra_tab,
                            src_rows, dst_tok)
    return out.reshape(_B, _S, _H)


# speedup vs baseline: 1.0138x; 1.0138x over previous
import functools

import jax
import jax.numpy as jnp
from jax import lax
from jax.experimental import pallas as pl
from jax.experimental.pallas import tpu as pltpu
from jax.experimental.pallas import tpu_sc as plsc

SOC_G = 17
SOC_L = 23

_B = 4
_S = 2048
_N = _B * _S
_H = 1024
_G = 1024
_LMAX = 16

_NC = 2
_NS = 16
_NW = _NC * _NS
_TPW = _N // _NW
_T = 32
_NCH = _TPW // _T
_K = 8
_R = 72
_NR = _N // 128
_RPB = _S // 128


def _f32(x):
    return x.astype(jnp.float32)


def _tc_body(ids_ref, lmf_ref, g_ref, lv_ref, w_ref, b_ref,
             extra_ref, srow_ref, dtok_ref):
    f32 = jnp.float32
    ids = ids_ref[...]
    lmf = lmf_ref[...]

    w = w_ref[...]
    bias = b_ref[...]
    pg = jnp.dot(g_ref[...], w, preferred_element_type=f32) + bias
    plv = jnp.dot(lv_ref[...], w, preferred_element_type=f32) + bias
    extra_ref[...] = jnp.concatenate([pg, plv, pg], axis=0)

    il = _f32(ids == SOC_L)
    ig = _f32(ids == SOC_G)

    m_i = lax.broadcasted_iota(jnp.int32, (128, 128), 0)
    m_j = lax.broadcasted_iota(jnp.int32, (128, 128), 1)
    m_incl = _f32(m_i <= m_j)
    m_last = _f32(m_i == 127)

    r_i = lax.broadcasted_iota(jnp.int32, (_NR, _NR), 0)
    c_i = lax.broadcasted_iota(jnp.int32, (_NR, _NR), 1)

    incl = jnp.dot(il, m_incl, preferred_element_type=f32)
    bsf = jnp.dot(incl, m_last, preferred_element_type=f32)
    m_carry = _f32((r_i // _RPB == c_i // _RPB) & (c_i < r_i))
    carry = jnp.dot(m_carry, bsf, preferred_element_type=f32)
    rank = incl + carry - 1.0

    sel = _f32(lax.broadcasted_iota(jnp.int32, (_NR, _B), 0) // _RPB
               == lax.broadcasted_iota(jnp.int32, (_NR, _B), 1))
    ones16_128 = jnp.ones((_LMAX, 128), f32)
    nv = jnp.dot(sel, jnp.dot(lmf, ones16_128, preferred_element_type=f32),
                 preferred_element_type=f32)
    inject = (il > 0.0) & (rank < nv)

    l_i = lax.broadcasted_iota(jnp.int32, (_LMAX, _LMAX), 0)
    l_j = lax.broadcasted_iota(jnp.int32, (_LMAX, _LMAX), 1)
    cm = jnp.dot(lmf, _f32(l_i <= l_j), preferred_element_type=f32)
    jidx = _f32(lax.broadcasted_iota(jnp.int32, (_B, _LMAX), 1))
    st_cols = [
        jnp.sum(lmf * _f32(cm == (r + 1)) * jidx, axis=1, keepdims=True)
        for r in range(_LMAX)
    ]
    st = jnp.concatenate(st_cols, axis=1)
    st64 = jnp.dot(sel, st, preferred_element_type=f32)

    rankc = jnp.clip(rank, 0.0, float(_LMAX - 1))
    slot = jnp.zeros_like(rank)
    for r in range(_LMAX):
        slot = slot + jnp.where(rankc == float(r), st64[:, r:r + 1], 0.0)

    bidx = _f32(lax.broadcasted_iota(jnp.int32, (_NR, 1), 0) // _RPB)
    extrarow = jnp.where(ig > 0.0, bidx,
                         float(_B) + bidx * float(_LMAX) + slot)
    ovr = ig + _f32(inject)

    incl2 = jnp.dot(ovr, m_incl, preferred_element_type=f32)
    bsf2 = jnp.dot(incl2, m_last, preferred_element_type=f32)
    m_pair = _f32((c_i == r_i - 1) & (r_i % 2 == 1))
    carry2 = jnp.dot(m_pair, bsf2, preferred_element_type=f32)
    oc = incl2 + carry2

    pos = _f32((lax.broadcasted_iota(jnp.int32, (_NR, 128), 0) % 2) * 128
               + lax.broadcasted_iota(jnp.int32, (_NR, 128), 1))
    ods, vls = [], []
    for k in range(_K):
        hit = ovr * _f32(oc == float(k + 1))
        ods.append(jnp.sum(hit * pos, axis=1, keepdims=True))
        vls.append(jnp.sum(hit * extrarow, axis=1, keepdims=True))
    ords = jnp.concatenate(ods, axis=1)
    vals = jnp.concatenate(vls, axis=1)

    pair = _f32(lax.broadcasted_iota(jnp.int32, (_NW, _NR), 1) // 2
                == lax.broadcasted_iota(jnp.int32, (_NW, _NR), 0))
    ordc = jnp.dot(pair, ords, preferred_element_type=f32)
    valc = jnp.dot(pair, vals, preferred_element_type=f32)
    cntc = jnp.dot(pair, bsf2[:, 0:_K], preferred_element_type=f32)

    kio = _f32(lax.broadcasted_iota(jnp.int32, (_NW, _K), 1))
    wio = _f32(lax.broadcasted_iota(jnp.int32, (_NW, _K), 0))
    valid = kio < cntc
    srow_ref[...] = jnp.where(valid, valc, 0.0).astype(jnp.int32)
    dtok_ref[...] = jnp.where(
        valid, wio * float(_TPW) + ordc, 0.0).astype(jnp.int32)


def _tc_bookkeeping(ids2d, lmf, gvec, lvec, proj_w, proj_b2d):
    return pl.pallas_call(
        _tc_body,
        out_shape=(
            jax.ShapeDtypeStruct((_R, _H), jnp.float32),
            jax.ShapeDtypeStruct((_NW, _K), jnp.int32),
            jax.ShapeDtypeStruct((_NW, _K), jnp.int32),
        ),
    )(ids2d, lmf, gvec, lvec, proj_w, proj_b2d)


@functools.cache
def _make_sc_gather():
    mesh = plsc.VectorSubcoreMesh(core_axis_name="c", subcore_axis_name="s",
                                  num_cores=_NC, num_subcores=_NS)

    @functools.partial(
        pl.kernel,
        out_type=jax.ShapeDtypeStruct((_N, _H), jnp.float32),
        mesh=mesh,
        scratch_types=[
            pltpu.VMEM((_TPW,), jnp.int32),
            pltpu.VMEM((56, _H), jnp.float32),
            pltpu.VMEM((56, _H), jnp.float32),
            pltpu.VMEM((_K,), jnp.int32),
            pltpu.VMEM((_K,), jnp.int32),
            pltpu.VMEM((_K, _H), jnp.float32),
            pltpu.SemaphoreType.DMA,
            pltpu.SemaphoreType.DMA,
            pltpu.SemaphoreType.DMA,
            pltpu.SemaphoreType.DMA,
            pltpu.SemaphoreType.DMA,
        ],
    )
    def sc_gather(ids_hbm, emb_hbm, extra_hbm, srow_hbm, dtok_hbm, out_hbm,
                  idx_v, buf0, buf1, srow_v, dtok_v, stag_v,
                  gs0, gs1, ss0, ss1, osem):
        wid = lax.axis_index("s") * _NC + lax.axis_index("c")
        base = wid * _TPW
        bufs = (buf0, buf1)
        gsems = (gs0, gs1)
        ssems = (ss0, ss1)
        steps = (56, 56, 56, 56, 32)
        offs = (0, 56, 112, 168, 224)
        nch = len(steps)
        nbuf = 2
        lag = 1
        pltpu.sync_copy(ids_hbm.at[pl.ds(base, _TPW)], idx_v)
        hrow = hdtk = None
        hg = [None] * nch
        hs = [None] * nch
        scatter_waited = [False] * nch
        for c in range(nch + lag):
            if c < nch:
                b = c % nbuf
                if c >= nbuf:
                    hs[c - nbuf].wait()
                    scatter_waited[c - nbuf] = True
                hg[c] = pltpu.async_copy(
                    emb_hbm.at[idx_v.at[pl.ds(offs[c], steps[c])]],
                    bufs[b].at[pl.ds(0, steps[c])], gsems[b])
            if c == 0:
                hrow = pltpu.async_copy(srow_hbm.at[wid], srow_v, osem)
                hdtk = pltpu.async_copy(dtok_hbm.at[wid], dtok_v, osem)
            d = c - lag
            if d >= 0:
                hg[d].wait()
                hs[d] = pltpu.async_copy(
                    bufs[d % nbuf].at[pl.ds(0, steps[d])],
                    out_hbm.at[pl.ds(base + offs[d], steps[d])],
                    ssems[d % nbuf])
        hrow.wait()
        hdtk.wait()
        hstag = pltpu.async_copy(extra_hbm.at[srow_v], stag_v, osem)
        for d in range(nch):
            if not scatter_waited[d]:
                hs[d].wait()
        hstag.wait()
        pltpu.async_copy(stag_v, out_hbm.at[dtok_v], osem).wait()

    return sc_gather


def kernel(input_ids, attention_mask, global_vec, local_vecs_padded,
           local_mask, emb_weight, proj_w, proj_b):
    del attention_mask
    ids = input_ids.astype(jnp.int32)

    extra_tab, src_rows, dst_tok = _tc_bookkeeping(
        ids.reshape(_NR, 128), local_mask.astype(jnp.float32), global_vec,
        local_vecs_padded.reshape(_B * _LMAX, _G), proj_w,
        proj_b.reshape(1, _H))

    out = _make_sc_gather()(ids.reshape(_N), emb_weight, extra_tab,
                            src_rows, dst_tok)
    return out.reshape(_B, _S, _H)
